# baseline (speedup 1.0000x reference)
import functools

import jax
import jax.numpy as jnp
from jax import lax
from jax.experimental import pallas as pl
from jax.experimental.pallas import tpu as pltpu
from jax.experimental.pallas import tpu_sc as plsc

N = 10000
D = 128
HID = 256
E = 320000

NC = 2
NS = 16
NW = NC * NS
CHUNK = 128
NPAD = 10240
RPT = NPAD // NS
NCHUNK = 80
NCIDX = NCHUNK + 1
EPT = NCHUNK * CHUNK
EPAD = NCIDX * CHUNK * NW
DEGW = 16
ZR = 64

@functools.cache
def _mesh():
    return plsc.VectorSubcoreMesh(core_axis_name="c", subcore_axis_name="s",
                                  num_cores=NC, num_subcores=NS)


def _deg_body(dst3_hbm, degp_hbm, acc, didx1, ones, zbuf):
    c = lax.axis_index("c")
    s = lax.axis_index("s")
    wid = c * NS + s
    ones16 = jnp.ones((16,), jnp.float32)
    zeros16 = jnp.zeros((16,), jnp.float32)

    for i in range(CHUNK):
        for k in range(D // 16):
            ones[i, k * 16:(k + 1) * 16] = ones16
    for i in range(ZR):
        for k in range(D // 16):
            zbuf[i, k * 16:(k + 1) * 16] = zeros16

    for b in range(RPT // ZR):
        pltpu.sync_copy(zbuf, acc.at[pl.ds(s * RPT + b * ZR, ZR)])
    plsc.subcore_barrier()

    def step(j, carry):
        pltpu.sync_copy(dst3_hbm.at[wid].at[j], didx1)
        pltpu.sync_copy(ones, acc.at[didx1], add=True)
        return carry

    lax.fori_loop(0, NCHUNK, step, 0)
    plsc.subcore_barrier()
    pltpu.sync_copy(acc.at[pl.ds(s * RPT, RPT)],
                    degp_hbm.at[pl.ds(c * NPAD + s * RPT, RPT)])


@functools.cache
def _deg_call():
    return pl.kernel(
        _deg_body,
        out_type=jax.ShapeDtypeStruct((NC * NPAD, D), jnp.float32),
        mesh=_mesh(),
        scratch_types=[
            pltpu.VMEM_SHARED((NPAD, D), jnp.float32),
            pltpu.VMEM((CHUNK,), jnp.int32),
            pltpu.VMEM((CHUNK, D), jnp.float32),
            pltpu.VMEM((ZR, D), jnp.float32),
        ],
    )


def _spmm_body(src3_hbm, dst3_hbm, v_hbm, out_hbm,
               acc, sidx, didx_a, rows_a, zbuf, sem_a):
    c = lax.axis_index("c")
    s = lax.axis_index("s")
    wid = c * NS + s
    zeros16 = jnp.zeros((16,), jnp.float32)

    for i in range(ZR):
        for k in range(D // 16):
            zbuf[i, k * 16:(k + 1) * 16] = zeros16

    for b in range(RPT // ZR):
        pltpu.sync_copy(zbuf, acc.at[pl.ds(s * RPT + b * ZR, ZR)])
    pltpu.sync_copy(src3_hbm.at[wid], sidx)
    plsc.subcore_barrier()

    def step(j, carry):
        pltpu.sync_copy(dst3_hbm.at[wid].at[j], didx_a)
        pltpu.async_copy(v_hbm.at[sidx.at[j]], rows_a, sem_a).wait()
        pltpu.sync_copy(rows_a, acc.at[didx_a], add=True)
        return carry

    lax.fori_loop(0, NCHUNK, step, 0)
    plsc.subcore_barrier()
    pltpu.sync_copy(acc.at[pl.ds(s * RPT, RPT)],
                    out_hbm.at[pl.ds(c * NPAD + s * RPT, RPT)])


@functools.cache
def _spmm_call():
    return pl.kernel(
        _spmm_body,
        out_type=jax.ShapeDtypeStruct((NC * NPAD, D), jnp.float32),
        mesh=_mesh(),
        scratch_types=[
            pltpu.VMEM_SHARED((NPAD, D), jnp.float32),
            pltpu.VMEM((NCIDX, CHUNK), jnp.int32),
            pltpu.VMEM((CHUNK,), jnp.int32),
            pltpu.VMEM((CHUNK, D), jnp.float32),
            pltpu.VMEM((ZR, D), jnp.float32),
            pltpu.SemaphoreType.DMA,
        ],
    )


RB = 1000
GRID = N // RB


def _tc1_body(dg0, dg1, x_ref, dinvb_ref, xs_ref):
    deg = dg0[0] + dg1[0] + 1.0
    dinvb = lax.rsqrt(deg)
    dinvb_ref[...] = dinvb
    xs_ref[...] = x_ref[...] * dinvb


_tc1_call = pl.pallas_call(
    _tc1_body,
    grid=(GRID,),
    in_specs=[
        pl.BlockSpec((1, RB, D), lambda i: (0, i, 0)),
        pl.BlockSpec((1, RB, D), lambda i: (1, i, 0)),
        pl.BlockSpec((RB, D), lambda i: (i, 0)),
    ],
    out_specs=[
        pl.BlockSpec((RB, D), lambda i: (i, 0)),
        pl.BlockSpec((RB, D), lambda i: (i, 0)),
    ],
    out_shape=[
        jax.ShapeDtypeStruct((N, D), jnp.float32),
        jax.ShapeDtypeStruct((N, D), jnp.float32),
    ],
)


def _tc2_body(p0, p1, xs, dinvb, w1, bias1, w2, gs_ref):
    y1 = dinvb[...] * (p0[0] + p1[0] + xs[...])
    h = jnp.dot(y1, w1[...], preferred_element_type=jnp.float32) + bias1[...]
    h = jnp.maximum(h, 0.0)
    g = jnp.dot(h, w2[...], preferred_element_type=jnp.float32)
    gs_ref[...] = g * dinvb[...]


_tc2_call = pl.pallas_call(
    _tc2_body,
    grid=(GRID,),
    in_specs=[
        pl.BlockSpec((1, RB, D), lambda i: (0, i, 0)),
        pl.BlockSpec((1, RB, D), lambda i: (1, i, 0)),
        pl.BlockSpec((RB, D), lambda i: (i, 0)),
        pl.BlockSpec((RB, D), lambda i: (i, 0)),
        pl.BlockSpec((D, HID), lambda i: (0, 0)),
        pl.BlockSpec((1, HID), lambda i: (0, 0)),
        pl.BlockSpec((HID, D), lambda i: (0, 0)),
    ],
    out_specs=[pl.BlockSpec((RB, D), lambda i: (i, 0))],
    out_shape=[jax.ShapeDtypeStruct((N, D), jnp.float32)],
)


def _tc3_body(q0, q1, gs, dinvb, bias2, out_ref):
    out_ref[...] = dinvb[...] * (q0[0] + q1[0] + gs[...]) + bias2[...]


_tc3_call = pl.pallas_call(
    _tc3_body,
    grid=(GRID,),
    in_specs=[
        pl.BlockSpec((1, RB, D), lambda i: (0, i, 0)),
        pl.BlockSpec((1, RB, D), lambda i: (1, i, 0)),
        pl.BlockSpec((RB, D), lambda i: (i, 0)),
        pl.BlockSpec((RB, D), lambda i: (i, 0)),
        pl.BlockSpec((1, D), lambda i: (0, 0)),
    ],
    out_specs=[pl.BlockSpec((RB, D), lambda i: (i, 0))],
    out_shape=[jax.ShapeDtypeStruct((N, D), jnp.float32)],
)


def kernel(x, edge_index, W1, b1, W2, b2):
    src = edge_index[0].astype(jnp.int32)
    dst = edge_index[1].astype(jnp.int32)
    npad_e = NW * EPT - E
    pad_dst = N + jnp.arange(npad_e, dtype=jnp.int32) % (NPAD - N)
    src3 = jnp.concatenate(
        [src, jnp.zeros((npad_e,), jnp.int32)]).reshape(NW, NCHUNK, CHUNK)
    dst3 = jnp.concatenate([dst, pad_dst]).reshape(NW, NCHUNK, CHUNK)
    src3 = jnp.concatenate(
        [src3, jnp.zeros((NW, 1, CHUNK), jnp.int32)], axis=1)
    dst3 = jnp.concatenate(
        [dst3, jnp.full((NW, 1, CHUNK), N, jnp.int32)], axis=1)

    degp = _deg_call()(dst3).reshape(NC, NPAD, D)
    dinvb, xs = _tc1_call(degp, degp, x)
    p = _spmm_call()(src3, dst3, xs).reshape(NC, NPAD, D)
    (gs,) = _tc2_call(p, p, xs, dinvb, W1, b1.reshape(1, HID), W2)
    q = _spmm_call()(src3, dst3, gs).reshape(NC, NPAD, D)
    (out,) = _tc3_call(q, q, gs, dinvb, b2.reshape(1, D))
    return out


# speedup vs baseline: 2.3079x; 2.3079x over previous
import functools

import jax
import jax.numpy as jnp
from jax import lax
from jax.experimental import pallas as pl
from jax.experimental.pallas import tpu as pltpu
from jax.experimental.pallas import tpu_sc as plsc

N = 10000
D = 128
HID = 256
E = 320000

NC = 2
NS = 16
NW = NC * NS
CHUNK = 128
NPAD = 10240
RPT = NPAD // NS
NCHUNK = 79
NCIDX = NCHUNK
EPT = NCHUNK * CHUNK
EPAD = NCIDX * CHUNK * NW
DEGW = 16
ZR = 64

@functools.cache
def _mesh():
    return plsc.VectorSubcoreMesh(core_axis_name="c", subcore_axis_name="s",
                                  num_cores=NC, num_subcores=NS)


def _deg_body(dst3_hbm, degp_hbm, acc, didx1, ones, zbuf):
    c = lax.axis_index("c")
    s = lax.axis_index("s")
    wid = c * NS + s
    ones16 = jnp.ones((16,), jnp.float32)
    zeros16 = jnp.zeros((16,), jnp.float32)

    for i in range(CHUNK):
        for k in range(D // 16):
            ones[i, k * 16:(k + 1) * 16] = ones16
    for i in range(ZR):
        for k in range(D // 16):
            zbuf[i, k * 16:(k + 1) * 16] = zeros16

    for b in range(RPT // ZR):
        pltpu.sync_copy(zbuf, acc.at[pl.ds(s * RPT + b * ZR, ZR)])
    plsc.subcore_barrier()

    def step(j, carry):
        pltpu.sync_copy(dst3_hbm.at[wid].at[j], didx1)
        pltpu.sync_copy(ones, acc.at[didx1], add=True)
        return carry

    lax.fori_loop(0, NCHUNK, step, 0)
    plsc.subcore_barrier()
    pltpu.sync_copy(acc.at[pl.ds(s * RPT, RPT)],
                    degp_hbm.at[pl.ds(c * NPAD + s * RPT, RPT)])


@functools.cache
def _deg_call():
    return pl.kernel(
        _deg_body,
        out_type=jax.ShapeDtypeStruct((NC * NPAD, D), jnp.float32),
        mesh=_mesh(),
        scratch_types=[
            pltpu.VMEM_SHARED((NPAD, D), jnp.float32),
            pltpu.VMEM((CHUNK,), jnp.int32),
            pltpu.VMEM((CHUNK, D), jnp.float32),
            pltpu.VMEM((ZR, D), jnp.float32),
        ],
    )


def _spmm_body(src3_hbm, dst3_hbm, v_hbm, out_hbm,
               acc, sidx, didx_a, rows_a, zbuf, sem_a):
    c = lax.axis_index("c")
    s = lax.axis_index("s")
    wid = c * NS + s
    zeros16 = jnp.zeros((16,), jnp.float32)

    for i in range(ZR):
        for k in range(D // 16):
            zbuf[i, k * 16:(k + 1) * 16] = zeros16

    for b in range(RPT // ZR):
        pltpu.sync_copy(zbuf, acc.at[pl.ds(s * RPT + b * ZR, ZR)])
    pltpu.sync_copy(src3_hbm.at[wid], sidx)
    plsc.subcore_barrier()

    def step(j, carry):
        pltpu.sync_copy(dst3_hbm.at[wid].at[j], didx_a)
        pltpu.async_copy(v_hbm.at[sidx.at[j]], rows_a, sem_a).wait()
        pltpu.sync_copy(rows_a, acc.at[didx_a], add=True)
        return carry

    lax.fori_loop(0, NCHUNK, step, 0)
    plsc.subcore_barrier()
    pltpu.sync_copy(acc.at[pl.ds(s * RPT, RPT)],
                    out_hbm.at[pl.ds(c * NPAD + s * RPT, RPT)])


@functools.cache
def _spmm_call():
    return pl.kernel(
        _spmm_body,
        out_type=jax.ShapeDtypeStruct((NC * NPAD, D), jnp.float32),
        mesh=_mesh(),
        scratch_types=[
            pltpu.VMEM_SHARED((NPAD, D), jnp.float32),
            pltpu.VMEM((NCIDX, CHUNK), jnp.int32),
            pltpu.VMEM((CHUNK,), jnp.int32),
            pltpu.VMEM((CHUNK, D), jnp.float32),
            pltpu.VMEM((ZR, D), jnp.float32),
            pltpu.SemaphoreType.DMA,
        ],
    )


RB = 1000
GRID = N // RB


def _tc1_body(dg0, dg1, x_ref, dinvb_ref, xs_ref):
    deg = dg0[0] + dg1[0] + 1.0
    dinvb = lax.rsqrt(deg)
    dinvb_ref[...] = dinvb
    xs_ref[...] = x_ref[...] * dinvb


_tc1_call = pl.pallas_call(
    _tc1_body,
    grid=(GRID,),
    in_specs=[
        pl.BlockSpec((1, RB, D), lambda i: (0, i, 0)),
        pl.BlockSpec((1, RB, D), lambda i: (1, i, 0)),
        pl.BlockSpec((RB, D), lambda i: (i, 0)),
    ],
    out_specs=[
        pl.BlockSpec((RB, D), lambda i: (i, 0)),
        pl.BlockSpec((RB, D), lambda i: (i, 0)),
    ],
    out_shape=[
        jax.ShapeDtypeStruct((N, D), jnp.float32),
        jax.ShapeDtypeStruct((N, D), jnp.float32),
    ],
)


def _tc2_body(p0, p1, xs, dinvb, w1, bias1, w2, gs_ref):
    y1 = dinvb[...] * (p0[0] + p1[0] + xs[...])
    h = jnp.dot(y1, w1[...], preferred_element_type=jnp.float32) + bias1[...]
    h = jnp.maximum(h, 0.0)
    g = jnp.dot(h, w2[...], preferred_element_type=jnp.float32)
    gs_ref[...] = g * dinvb[...]


_tc2_call = pl.pallas_call(
    _tc2_body,
    grid=(GRID,),
    in_specs=[
        pl.BlockSpec((1, RB, D), lambda i: (0, i, 0)),
        pl.BlockSpec((1, RB, D), lambda i: (1, i, 0)),
        pl.BlockSpec((RB, D), lambda i: (i, 0)),
        pl.BlockSpec((RB, D), lambda i: (i, 0)),
        pl.BlockSpec((D, HID), lambda i: (0, 0)),
        pl.BlockSpec((1, HID), lambda i: (0, 0)),
        pl.BlockSpec((HID, D), lambda i: (0, 0)),
    ],
    out_specs=[pl.BlockSpec((RB, D), lambda i: (i, 0))],
    out_shape=[jax.ShapeDtypeStruct((N, D), jnp.float32)],
)


def _tc3_body(q0, q1, gs, dinvb, bias2, out_ref):
    out_ref[...] = dinvb[...] * (q0[0] + q1[0] + gs[...]) + bias2[...]


_tc3_call = pl.pallas_call(
    _tc3_body,
    grid=(GRID,),
    in_specs=[
        pl.BlockSpec((1, RB, D), lambda i: (0, i, 0)),
        pl.BlockSpec((1, RB, D), lambda i: (1, i, 0)),
        pl.BlockSpec((RB, D), lambda i: (i, 0)),
        pl.BlockSpec((RB, D), lambda i: (i, 0)),
        pl.BlockSpec((1, D), lambda i: (0, 0)),
    ],
    out_specs=[pl.BlockSpec((RB, D), lambda i: (i, 0))],
    out_shape=[jax.ShapeDtypeStruct((N, D), jnp.float32)],
)


def kernel(x, edge_index, W1, b1, W2, b2):
    src = edge_index[0].astype(jnp.int32)
    dst = edge_index[1].astype(jnp.int32)
    npad_e = NW * EPT - E
    pad_iota = jnp.arange(npad_e, dtype=jnp.int32)
    pad_src = pad_iota % N
    pad_dst = N + pad_iota % (NPAD - N)
    src3 = jnp.concatenate([src, pad_src]).reshape(NW, NCHUNK, CHUNK)
    dst3 = jnp.concatenate([dst, pad_dst]).reshape(NW, NCHUNK, CHUNK)

    degp = _deg_call()(dst3).reshape(NC, NPAD, D)
    dinvb, xs = _tc1_call(degp, degp, x)
    p = _spmm_call()(src3, dst3, xs).reshape(NC, NPAD, D)
    (gs,) = _tc2_call(p, p, xs, dinvb, W1, b1.reshape(1, HID), W2)
    q = _spmm_call()(src3, dst3, gs).reshape(NC, NPAD, D)
    (out,) = _tc3_call(q, q, gs, dinvb, b2.reshape(1, D))
    return out


# speedup vs baseline: 2.8953x; 1.2545x over previous
import functools

import jax
import jax.numpy as jnp
from jax import lax
from jax.experimental import pallas as pl
from jax.experimental.pallas import tpu as pltpu
from jax.experimental.pallas import tpu_sc as plsc

N = 10000
D = 128
HID = 256
E = 320000

NC = 2
NS = 16
NW = NC * NS
CHUNK = 128
NPAD = 10240
RPT = NPAD // NS
NCHUNK = 80
NCIDX = NCHUNK + 1
EPT = NCHUNK * CHUNK
EPAD = NCIDX * CHUNK * NW
DEGW = 16
ZR = 64

@functools.cache
def _mesh():
    return plsc.VectorSubcoreMesh(core_axis_name="c", subcore_axis_name="s",
                                  num_cores=NC, num_subcores=NS)


def _deg_body(dst3_hbm, degp_hbm, acc, didx1, ones, zbuf):
    c = lax.axis_index("c")
    s = lax.axis_index("s")
    wid = c * NS + s
    ones16 = jnp.ones((16,), jnp.float32)
    zeros16 = jnp.zeros((16,), jnp.float32)

    for i in range(CHUNK):
        for k in range(D // 16):
            ones[i, k * 16:(k + 1) * 16] = ones16
    for i in range(ZR):
        for k in range(D // 16):
            zbuf[i, k * 16:(k + 1) * 16] = zeros16

    for b in range(RPT // ZR):
        pltpu.sync_copy(zbuf, acc.at[pl.ds(s * RPT + b * ZR, ZR)])
    plsc.subcore_barrier()

    def step(j, carry):
        pltpu.sync_copy(dst3_hbm.at[wid].at[j], didx1)
        pltpu.sync_copy(ones, acc.at[didx1], add=True)
        return carry

    lax.fori_loop(0, NCHUNK, step, 0)
    plsc.subcore_barrier()
    pltpu.sync_copy(acc.at[pl.ds(s * RPT, RPT)],
                    degp_hbm.at[pl.ds(c * NPAD + s * RPT, RPT)])


@functools.cache
def _deg_call():
    return pl.kernel(
        _deg_body,
        out_type=jax.ShapeDtypeStruct((NC * NPAD, D), jnp.float32),
        mesh=_mesh(),
        scratch_types=[
            pltpu.VMEM_SHARED((NPAD, D), jnp.float32),
            pltpu.VMEM((CHUNK,), jnp.int32),
            pltpu.VMEM((CHUNK, D), jnp.float32),
            pltpu.VMEM((ZR, D), jnp.float32),
        ],
    )


def _spmm_body(src3_hbm, dst3_hbm, v_hbm, out_hbm, acc,
               sidx_a, sidx_b, didx_a, didx_b, rows_a, rows_b, zbuf,
               sem_a, sem_b):
    c = lax.axis_index("c")
    s = lax.axis_index("s")
    wid = c * NS + s
    zeros16 = jnp.zeros((16,), jnp.float32)

    for i in range(ZR):
        for k in range(D // 16):
            zbuf[i, k * 16:(k + 1) * 16] = zeros16

    for b in range(RPT // ZR):
        pltpu.sync_copy(zbuf, acc.at[pl.ds(s * RPT + b * ZR, ZR)])
    plsc.subcore_barrier()

    pltpu.sync_copy(src3_hbm.at[wid].at[0], sidx_a)
    pltpu.sync_copy(dst3_hbm.at[wid].at[0], didx_a)
    pltpu.async_copy(v_hbm.at[sidx_a], rows_a, sem_a)

    def step(k, carry):
        j0 = 2 * k
        pltpu.sync_copy(src3_hbm.at[wid].at[j0 + 1], sidx_b)
        pltpu.sync_copy(dst3_hbm.at[wid].at[j0 + 1], didx_b)
        pltpu.async_copy(v_hbm.at[sidx_b], rows_b, sem_b)
        pltpu.make_async_copy(v_hbm.at[sidx_a], rows_a, sem_a).wait()
        pltpu.sync_copy(rows_a, acc.at[didx_a], add=True)
        pltpu.sync_copy(src3_hbm.at[wid].at[j0 + 2], sidx_a)
        pltpu.sync_copy(dst3_hbm.at[wid].at[j0 + 2], didx_a)
        pltpu.async_copy(v_hbm.at[sidx_a], rows_a, sem_a)
        pltpu.make_async_copy(v_hbm.at[sidx_b], rows_b, sem_b).wait()
        pltpu.sync_copy(rows_b, acc.at[didx_b], add=True)
        return carry

    lax.fori_loop(0, NCHUNK // 2, step, 0)
    pltpu.make_async_copy(v_hbm.at[sidx_a], rows_a, sem_a).wait()
    plsc.subcore_barrier()
    pltpu.sync_copy(acc.at[pl.ds(s * RPT, RPT)],
                    out_hbm.at[pl.ds(c * NPAD + s * RPT, RPT)])


@functools.cache
def _spmm_call():
    return pl.kernel(
        _spmm_body,
        out_type=jax.ShapeDtypeStruct((NC * NPAD, D), jnp.float32),
        mesh=_mesh(),
        scratch_types=[
            pltpu.VMEM_SHARED((NPAD, D), jnp.float32),
            pltpu.VMEM((CHUNK,), jnp.int32),
            pltpu.VMEM((CHUNK,), jnp.int32),
            pltpu.VMEM((CHUNK,), jnp.int32),
            pltpu.VMEM((CHUNK,), jnp.int32),
            pltpu.VMEM((CHUNK, D), jnp.float32),
            pltpu.VMEM((CHUNK, D), jnp.float32),
            pltpu.VMEM((ZR, D), jnp.float32),
            pltpu.SemaphoreType.DMA,
            pltpu.SemaphoreType.DMA,
        ],
    )


RB = 1000
GRID = N // RB


def _tc1_body(dg0, dg1, x_ref, dinvb_ref, xs_ref):
    deg = dg0[0] + dg1[0] + 1.0
    dinvb = lax.rsqrt(deg)
    dinvb_ref[...] = dinvb
    xs_ref[...] = x_ref[...] * dinvb


_tc1_call = pl.pallas_call(
    _tc1_body,
    grid=(GRID,),
    in_specs=[
        pl.BlockSpec((1, RB, D), lambda i: (0, i, 0)),
        pl.BlockSpec((1, RB, D), lambda i: (1, i, 0)),
        pl.BlockSpec((RB, D), lambda i: (i, 0)),
    ],
    out_specs=[
        pl.BlockSpec((RB, D), lambda i: (i, 0)),
        pl.BlockSpec((RB, D), lambda i: (i, 0)),
    ],
    out_shape=[
        jax.ShapeDtypeStruct((N, D), jnp.float32),
        jax.ShapeDtypeStruct((N, D), jnp.float32),
    ],
)


def _tc2_body(p0, p1, xs, dinvb, w1, bias1, w2, gs_ref):
    y1 = dinvb[...] * (p0[0] + p1[0] + xs[...])
    h = jnp.dot(y1, w1[...], preferred_element_type=jnp.float32) + bias1[...]
    h = jnp.maximum(h, 0.0)
    g = jnp.dot(h, w2[...], preferred_element_type=jnp.float32)
    gs_ref[...] = g * dinvb[...]


_tc2_call = pl.pallas_call(
    _tc2_body,
    grid=(GRID,),
    in_specs=[
        pl.BlockSpec((1, RB, D), lambda i: (0, i, 0)),
        pl.BlockSpec((1, RB, D), lambda i: (1, i, 0)),
        pl.BlockSpec((RB, D), lambda i: (i, 0)),
        pl.BlockSpec((RB, D), lambda i: (i, 0)),
        pl.BlockSpec((D, HID), lambda i: (0, 0)),
        pl.BlockSpec((1, HID), lambda i: (0, 0)),
        pl.BlockSpec((HID, D), lambda i: (0, 0)),
    ],
    out_specs=[pl.BlockSpec((RB, D), lambda i: (i, 0))],
    out_shape=[jax.ShapeDtypeStruct((N, D), jnp.float32)],
)


def _tc3_body(q0, q1, gs, dinvb, bias2, out_ref):
    out_ref[...] = dinvb[...] * (q0[0] + q1[0] + gs[...]) + bias2[...]


_tc3_call = pl.pallas_call(
    _tc3_body,
    grid=(GRID,),
    in_specs=[
        pl.BlockSpec((1, RB, D), lambda i: (0, i, 0)),
        pl.BlockSpec((1, RB, D), lambda i: (1, i, 0)),
        pl.BlockSpec((RB, D), lambda i: (i, 0)),
        pl.BlockSpec((RB, D), lambda i: (i, 0)),
        pl.BlockSpec((1, D), lambda i: (0, 0)),
    ],
    out_specs=[pl.BlockSpec((RB, D), lambda i: (i, 0))],
    out_shape=[jax.ShapeDtypeStruct((N, D), jnp.float32)],
)


def kernel(x, edge_index, W1, b1, W2, b2):
    src = edge_index[0].astype(jnp.int32)
    dst = edge_index[1].astype(jnp.int32)
    npad_e = NW * EPT - E
    pad_iota = jnp.arange(npad_e, dtype=jnp.int32)
    pad_src = pad_iota % N
    pad_dst = N + pad_iota % (NPAD - N)
    src3 = jnp.concatenate([src, pad_src]).reshape(NW, NCHUNK, CHUNK)
    dst3 = jnp.concatenate([dst, pad_dst]).reshape(NW, NCHUNK, CHUNK)
    dum = jnp.arange(NW * CHUNK, dtype=jnp.int32).reshape(NW, 1, CHUNK)
    src3 = jnp.concatenate([src3, dum % N], axis=1)
    dst3 = jnp.concatenate([dst3, N + dum % (NPAD - N)], axis=1)

    degp = _deg_call()(dst3).reshape(NC, NPAD, D)
    dinvb, xs = _tc1_call(degp, degp, x)
    p = _spmm_call()(src3, dst3, xs).reshape(NC, NPAD, D)
    (gs,) = _tc2_call(p, p, xs, dinvb, W1, b1.reshape(1, HID), W2)
    q = _spmm_call()(src3, dst3, gs).reshape(NC, NPAD, D)
    (out,) = _tc3_call(q, q, gs, dinvb, b2.reshape(1, D))
    return out


# speedup vs baseline: 3.1200x; 1.0776x over previous
import functools

import jax
import jax.numpy as jnp
from jax import lax
from jax.experimental import pallas as pl
from jax.experimental.pallas import tpu as pltpu
from jax.experimental.pallas import tpu_sc as plsc

N = 10000
D = 128
HID = 256
E = 320000

NC = 2
NS = 16
NW = NC * NS
CHUNK = 128
NPAD = 10240
RPT = NPAD // NS
NCHUNK = 80
NCIDX = NCHUNK + 1
EPT = NCHUNK * CHUNK
EPAD = NCIDX * CHUNK * NW
DEGW = 16
ZR = 64

@functools.cache
def _mesh():
    return plsc.VectorSubcoreMesh(core_axis_name="c", subcore_axis_name="s",
                                  num_cores=NC, num_subcores=NS)


def _deg_body(dst3_hbm, degp_hbm, acc, didx_a, didx_b, ones, zbuf,
              sem_a, sem_b):
    c = lax.axis_index("c")
    s = lax.axis_index("s")
    wid = c * NS + s
    ones16 = jnp.ones((16,), jnp.float32)
    zeros16 = jnp.zeros((16,), jnp.float32)

    for i in range(CHUNK):
        for k in range(D // 16):
            ones[i, k * 16:(k + 1) * 16] = ones16
    for i in range(ZR):
        for k in range(D // 16):
            zbuf[i, k * 16:(k + 1) * 16] = zeros16

    for b in range(RPT // ZR):
        pltpu.sync_copy(zbuf, acc.at[pl.ds(s * RPT + b * ZR, ZR)])
    plsc.subcore_barrier()

    pltpu.sync_copy(dst3_hbm.at[wid].at[0], didx_a)

    def step(k, carry):
        j0 = 2 * k
        pltpu.async_copy(ones, acc.at[didx_a], sem_a, add=True)
        pltpu.sync_copy(dst3_hbm.at[wid].at[j0 + 1], didx_b)
        pltpu.make_async_copy(ones, acc.at[didx_a], sem_a).wait()
        pltpu.async_copy(ones, acc.at[didx_b], sem_b, add=True)
        pltpu.sync_copy(dst3_hbm.at[wid].at[j0 + 2], didx_a)
        pltpu.make_async_copy(ones, acc.at[didx_b], sem_b).wait()
        return carry

    lax.fori_loop(0, NCHUNK // 2, step, 0)
    plsc.subcore_barrier()
    pltpu.sync_copy(acc.at[pl.ds(s * RPT, RPT)],
                    degp_hbm.at[pl.ds(c * NPAD + s * RPT, RPT)])


@functools.cache
def _deg_call():
    return pl.kernel(
        _deg_body,
        out_type=jax.ShapeDtypeStruct((NC * NPAD, D), jnp.float32),
        mesh=_mesh(),
        scratch_types=[
            pltpu.VMEM_SHARED((NPAD, D), jnp.float32),
            pltpu.VMEM((CHUNK,), jnp.int32),
            pltpu.VMEM((CHUNK,), jnp.int32),
            pltpu.VMEM((CHUNK, D), jnp.float32),
            pltpu.VMEM((ZR, D), jnp.float32),
            pltpu.SemaphoreType.DMA,
            pltpu.SemaphoreType.DMA,
        ],
    )


def _spmm_body(src3_hbm, dst3_hbm, v_hbm, out_hbm, acc,
               sidx_a, sidx_b, didx_a, didx_b, rows_a, rows_b, zbuf,
               sem_a, sem_b):
    c = lax.axis_index("c")
    s = lax.axis_index("s")
    wid = c * NS + s
    zeros16 = jnp.zeros((16,), jnp.float32)

    for i in range(ZR):
        for k in range(D // 16):
            zbuf[i, k * 16:(k + 1) * 16] = zeros16

    for b in range(RPT // ZR):
        pltpu.sync_copy(zbuf, acc.at[pl.ds(s * RPT + b * ZR, ZR)])
    plsc.subcore_barrier()

    pltpu.sync_copy(src3_hbm.at[wid].at[0], sidx_a)
    pltpu.sync_copy(dst3_hbm.at[wid].at[0], didx_a)
    pltpu.async_copy(v_hbm.at[sidx_a], rows_a, sem_a)

    def step(k, carry):
        j0 = 2 * k
        pltpu.sync_copy(src3_hbm.at[wid].at[j0 + 1], sidx_b)
        pltpu.sync_copy(dst3_hbm.at[wid].at[j0 + 1], didx_b)
        pltpu.async_copy(v_hbm.at[sidx_b], rows_b, sem_b)
        pltpu.make_async_copy(v_hbm.at[sidx_a], rows_a, sem_a).wait()
        pltpu.sync_copy(rows_a, acc.at[didx_a], add=True)
        pltpu.sync_copy(src3_hbm.at[wid].at[j0 + 2], sidx_a)
        pltpu.sync_copy(dst3_hbm.at[wid].at[j0 + 2], didx_a)
        pltpu.async_copy(v_hbm.at[sidx_a], rows_a, sem_a)
        pltpu.make_async_copy(v_hbm.at[sidx_b], rows_b, sem_b).wait()
        pltpu.sync_copy(rows_b, acc.at[didx_b], add=True)
        return carry

    lax.fori_loop(0, NCHUNK // 2, step, 0)
    pltpu.make_async_copy(v_hbm.at[sidx_a], rows_a, sem_a).wait()
    plsc.subcore_barrier()
    pltpu.sync_copy(acc.at[pl.ds(s * RPT, RPT)],
                    out_hbm.at[pl.ds(c * NPAD + s * RPT, RPT)])


@functools.cache
def _spmm_call():
    return pl.kernel(
        _spmm_body,
        out_type=jax.ShapeDtypeStruct((NC * NPAD, D), jnp.float32),
        mesh=_mesh(),
        scratch_types=[
            pltpu.VMEM_SHARED((NPAD, D), jnp.float32),
            pltpu.VMEM((CHUNK,), jnp.int32),
            pltpu.VMEM((CHUNK,), jnp.int32),
            pltpu.VMEM((CHUNK,), jnp.int32),
            pltpu.VMEM((CHUNK,), jnp.int32),
            pltpu.VMEM((CHUNK, D), jnp.float32),
            pltpu.VMEM((CHUNK, D), jnp.float32),
            pltpu.VMEM((ZR, D), jnp.float32),
            pltpu.SemaphoreType.DMA,
            pltpu.SemaphoreType.DMA,
        ],
    )


RB = 1000
GRID = N // RB


def _tc1_body(dg0, dg1, x_ref, dinvb_ref, xs_ref):
    deg = dg0[0] + dg1[0] + 1.0
    dinvb = lax.rsqrt(deg)
    dinvb_ref[...] = dinvb
    xs_ref[...] = x_ref[...] * dinvb


_tc1_call = pl.pallas_call(
    _tc1_body,
    grid=(GRID,),
    in_specs=[
        pl.BlockSpec((1, RB, D), lambda i: (0, i, 0)),
        pl.BlockSpec((1, RB, D), lambda i: (1, i, 0)),
        pl.BlockSpec((RB, D), lambda i: (i, 0)),
    ],
    out_specs=[
        pl.BlockSpec((RB, D), lambda i: (i, 0)),
        pl.BlockSpec((RB, D), lambda i: (i, 0)),
    ],
    out_shape=[
        jax.ShapeDtypeStruct((N, D), jnp.float32),
        jax.ShapeDtypeStruct((N, D), jnp.float32),
    ],
)


def _tc2_body(p0, p1, xs, dinvb, w1, bias1, w2, gs_ref):
    y1 = dinvb[...] * (p0[0] + p1[0] + xs[...])
    h = jnp.dot(y1, w1[...], preferred_element_type=jnp.float32) + bias1[...]
    h = jnp.maximum(h, 0.0)
    g = jnp.dot(h, w2[...], preferred_element_type=jnp.float32)
    gs_ref[...] = g * dinvb[...]


_tc2_call = pl.pallas_call(
    _tc2_body,
    grid=(GRID,),
    in_specs=[
        pl.BlockSpec((1, RB, D), lambda i: (0, i, 0)),
        pl.BlockSpec((1, RB, D), lambda i: (1, i, 0)),
        pl.BlockSpec((RB, D), lambda i: (i, 0)),
        pl.BlockSpec((RB, D), lambda i: (i, 0)),
        pl.BlockSpec((D, HID), lambda i: (0, 0)),
        pl.BlockSpec((1, HID), lambda i: (0, 0)),
        pl.BlockSpec((HID, D), lambda i: (0, 0)),
    ],
    out_specs=[pl.BlockSpec((RB, D), lambda i: (i, 0))],
    out_shape=[jax.ShapeDtypeStruct((N, D), jnp.float32)],
)


def _tc3_body(q0, q1, gs, dinvb, bias2, out_ref):
    out_ref[...] = dinvb[...] * (q0[0] + q1[0] + gs[...]) + bias2[...]


_tc3_call = pl.pallas_call(
    _tc3_body,
    grid=(GRID,),
    in_specs=[
        pl.BlockSpec((1, RB, D), lambda i: (0, i, 0)),
        pl.BlockSpec((1, RB, D), lambda i: (1, i, 0)),
        pl.BlockSpec((RB, D), lambda i: (i, 0)),
        pl.BlockSpec((RB, D), lambda i: (i, 0)),
        pl.BlockSpec((1, D), lambda i: (0, 0)),
    ],
    out_specs=[pl.BlockSpec((RB, D), lambda i: (i, 0))],
    out_shape=[jax.ShapeDtypeStruct((N, D), jnp.float32)],
)


def kernel(x, edge_index, W1, b1, W2, b2):
    src = edge_index[0].astype(jnp.int32)
    dst = edge_index[1].astype(jnp.int32)
    npad_e = NW * EPT - E
    pad_iota = jnp.arange(npad_e, dtype=jnp.int32)
    pad_src = pad_iota % N
    pad_dst = N + pad_iota % (NPAD - N)
    src3 = jnp.concatenate([src, pad_src]).reshape(NW, NCHUNK, CHUNK)
    dst3 = jnp.concatenate([dst, pad_dst]).reshape(NW, NCHUNK, CHUNK)
    dum = jnp.arange(NW * CHUNK, dtype=jnp.int32).reshape(NW, 1, CHUNK)
    src3 = jnp.concatenate([src3, dum % N], axis=1)
    dst3 = jnp.concatenate([dst3, N + dum % (NPAD - N)], axis=1)

    degp = _deg_call()(dst3).reshape(NC, NPAD, D)
    dinvb, xs = _tc1_call(degp, degp, x)
    p = _spmm_call()(src3, dst3, xs).reshape(NC, NPAD, D)
    (gs,) = _tc2_call(p, p, xs, dinvb, W1, b1.reshape(1, HID), W2)
    q = _spmm_call()(src3, dst3, gs).reshape(NC, NPAD, D)
    (out,) = _tc3_call(q, q, gs, dinvb, b2.reshape(1, D))
    return out


# speedup vs baseline: 3.6855x; 1.1813x over previous
import functools

import jax
import jax.numpy as jnp
from jax import lax
from jax.experimental import pallas as pl
from jax.experimental.pallas import tpu as pltpu
from jax.experimental.pallas import tpu_sc as plsc

N = 10000
D = 128
HID = 256
E = 320000

NC = 2
NS = 16
NW = NC * NS
CHUNK = 128
NPAD = 10240
RPT = NPAD // NS
NCHUNK = 80
NCIDX = NCHUNK + 1
EPT = NCHUNK * CHUNK
EPAD = NCIDX * CHUNK * NW
DEGW = 16
ZR = 64

@functools.cache
def _mesh():
    return plsc.VectorSubcoreMesh(core_axis_name="c", subcore_axis_name="s",
                                  num_cores=NC, num_subcores=NS)


def _deg_body(dst3_hbm, zeros_hbm, ones_hbm, degp_hbm, acc,
              didx_a, didx_b, ones, sem_a, sem_b):
    c = lax.axis_index("c")
    s = lax.axis_index("s")
    wid = c * NS + s

    pltpu.sync_copy(zeros_hbm.at[pl.ds(s * RPT, RPT)],
                    acc.at[pl.ds(s * RPT, RPT)])
    pltpu.sync_copy(ones_hbm, ones)
    plsc.subcore_barrier()

    pltpu.sync_copy(dst3_hbm.at[wid].at[0], didx_a)

    def step(k, carry):
        j0 = 2 * k
        pltpu.async_copy(ones, acc.at[didx_a], sem_a, add=True)
        pltpu.sync_copy(dst3_hbm.at[wid].at[j0 + 1], didx_b)
        pltpu.make_async_copy(ones, acc.at[didx_a], sem_a).wait()
        pltpu.async_copy(ones, acc.at[didx_b], sem_b, add=True)
        pltpu.sync_copy(dst3_hbm.at[wid].at[j0 + 2], didx_a)
        pltpu.make_async_copy(ones, acc.at[didx_b], sem_b).wait()
        return carry

    lax.fori_loop(0, NCHUNK // 2, step, 0)
    plsc.subcore_barrier()
    pltpu.sync_copy(acc.at[pl.ds(s * RPT, RPT)],
                    degp_hbm.at[pl.ds(c * NPAD + s * RPT, RPT)])


@functools.cache
def _deg_call():
    return pl.kernel(
        _deg_body,
        out_type=jax.ShapeDtypeStruct((NC * NPAD, D), jnp.float32),
        mesh=_mesh(),
        scratch_types=[
            pltpu.VMEM_SHARED((NPAD, D), jnp.float32),
            pltpu.VMEM((CHUNK,), jnp.int32),
            pltpu.VMEM((CHUNK,), jnp.int32),
            pltpu.VMEM((CHUNK, D), jnp.float32),
            pltpu.SemaphoreType.DMA,
            pltpu.SemaphoreType.DMA,
        ],
    )


def _spmm_body(src3_hbm, dst3_hbm, v_hbm, zeros_hbm, out_hbm, acc,
               sidx, didx_a, didx_b, rows_a, rows_b, sem_a, sem_b):
    c = lax.axis_index("c")
    s = lax.axis_index("s")
    wid = c * NS + s

    pltpu.sync_copy(zeros_hbm.at[pl.ds(s * RPT, RPT)],
                    acc.at[pl.ds(s * RPT, RPT)])
    pltpu.sync_copy(src3_hbm.at[wid], sidx)
    plsc.subcore_barrier()

    pltpu.sync_copy(dst3_hbm.at[wid].at[0], didx_a)
    pltpu.async_copy(v_hbm.at[sidx.at[0]], rows_a, sem_a)

    def step(k, carry):
        j0 = 2 * k
        pltpu.async_copy(v_hbm.at[sidx.at[j0 + 1]], rows_b, sem_b)
        pltpu.sync_copy(dst3_hbm.at[wid].at[j0 + 1], didx_b)
        pltpu.make_async_copy(v_hbm.at[sidx.at[j0]], rows_a, sem_a).wait()
        pltpu.sync_copy(rows_a, acc.at[didx_a], add=True)
        pltpu.async_copy(v_hbm.at[sidx.at[j0 + 2]], rows_a, sem_a)
        pltpu.sync_copy(dst3_hbm.at[wid].at[j0 + 2], didx_a)
        pltpu.make_async_copy(v_hbm.at[sidx.at[j0 + 1]], rows_b, sem_b).wait()
        pltpu.sync_copy(rows_b, acc.at[didx_b], add=True)
        return carry

    lax.fori_loop(0, NCHUNK // 2, step, 0)
    pltpu.make_async_copy(v_hbm.at[sidx.at[NCHUNK]], rows_a, sem_a).wait()
    plsc.subcore_barrier()
    pltpu.sync_copy(acc.at[pl.ds(s * RPT, RPT)],
                    out_hbm.at[pl.ds(c * NPAD + s * RPT, RPT)])


@functools.cache
def _spmm_call():
    return pl.kernel(
        _spmm_body,
        out_type=jax.ShapeDtypeStruct((NC * NPAD, D), jnp.float32),
        mesh=_mesh(),
        scratch_types=[
            pltpu.VMEM_SHARED((NPAD, D), jnp.float32),
            pltpu.VMEM((NCIDX, CHUNK), jnp.int32),
            pltpu.VMEM((CHUNK,), jnp.int32),
            pltpu.VMEM((CHUNK,), jnp.int32),
            pltpu.VMEM((CHUNK, D), jnp.float32),
            pltpu.VMEM((CHUNK, D), jnp.float32),
            pltpu.SemaphoreType.DMA,
            pltpu.SemaphoreType.DMA,
        ],
    )


RB = 1000
GRID = N // RB


def _tc1_body(dg0, dg1, x_ref, dinvb_ref, xs_ref):
    deg = dg0[0] + dg1[0] + 1.0
    dinvb = lax.rsqrt(deg)
    dinvb_ref[...] = dinvb
    xs_ref[...] = x_ref[...] * dinvb


_tc1_call = pl.pallas_call(
    _tc1_body,
    grid=(GRID,),
    in_specs=[
        pl.BlockSpec((1, RB, D), lambda i: (0, i, 0)),
        pl.BlockSpec((1, RB, D), lambda i: (1, i, 0)),
        pl.BlockSpec((RB, D), lambda i: (i, 0)),
    ],
    out_specs=[
        pl.BlockSpec((RB, D), lambda i: (i, 0)),
        pl.BlockSpec((RB, D), lambda i: (i, 0)),
    ],
    out_shape=[
        jax.ShapeDtypeStruct((N, D), jnp.float32),
        jax.ShapeDtypeStruct((N, D), jnp.float32),
    ],
)


def _tc2_body(p0, p1, xs, dinvb, w1, bias1, w2, gs_ref):
    y1 = dinvb[...] * (p0[0] + p1[0] + xs[...])
    h = jnp.dot(y1, w1[...], preferred_element_type=jnp.float32) + bias1[...]
    h = jnp.maximum(h, 0.0)
    g = jnp.dot(h, w2[...], preferred_element_type=jnp.float32)
    gs_ref[...] = g * dinvb[...]


_tc2_call = pl.pallas_call(
    _tc2_body,
    grid=(GRID,),
    in_specs=[
        pl.BlockSpec((1, RB, D), lambda i: (0, i, 0)),
        pl.BlockSpec((1, RB, D), lambda i: (1, i, 0)),
        pl.BlockSpec((RB, D), lambda i: (i, 0)),
        pl.BlockSpec((RB, D), lambda i: (i, 0)),
        pl.BlockSpec((D, HID), lambda i: (0, 0)),
        pl.BlockSpec((1, HID), lambda i: (0, 0)),
        pl.BlockSpec((HID, D), lambda i: (0, 0)),
    ],
    out_specs=[pl.BlockSpec((RB, D), lambda i: (i, 0))],
    out_shape=[jax.ShapeDtypeStruct((N, D), jnp.float32)],
)


def _tc3_body(q0, q1, gs, dinvb, bias2, out_ref):
    out_ref[...] = dinvb[...] * (q0[0] + q1[0] + gs[...]) + bias2[...]


_tc3_call = pl.pallas_call(
    _tc3_body,
    grid=(GRID,),
    in_specs=[
        pl.BlockSpec((1, RB, D), lambda i: (0, i, 0)),
        pl.BlockSpec((1, RB, D), lambda i: (1, i, 0)),
        pl.BlockSpec((RB, D), lambda i: (i, 0)),
        pl.BlockSpec((RB, D), lambda i: (i, 0)),
        pl.BlockSpec((1, D), lambda i: (0, 0)),
    ],
    out_specs=[pl.BlockSpec((RB, D), lambda i: (i, 0))],
    out_shape=[jax.ShapeDtypeStruct((N, D), jnp.float32)],
)


def kernel(x, edge_index, W1, b1, W2, b2):
    src = edge_index[0].astype(jnp.int32)
    dst = edge_index[1].astype(jnp.int32)
    npad_e = NW * EPT - E
    pad_iota = jnp.arange(npad_e, dtype=jnp.int32)
    pad_src = pad_iota % N
    pad_dst = N + pad_iota % (NPAD - N)
    src3 = jnp.concatenate([src, pad_src]).reshape(NW, NCHUNK, CHUNK)
    dst3 = jnp.concatenate([dst, pad_dst]).reshape(NW, NCHUNK, CHUNK)
    dum = jnp.arange(NW * CHUNK, dtype=jnp.int32).reshape(NW, 1, CHUNK)
    src3 = jnp.concatenate([src3, dum % N], axis=1)
    dst3 = jnp.concatenate([dst3, N + dum % (NPAD - N)], axis=1)

    zeros_c = jnp.zeros((NPAD, D), jnp.float32)
    ones_c = jnp.ones((CHUNK, D), jnp.float32)
    degp = _deg_call()(dst3, zeros_c, ones_c).reshape(NC, NPAD, D)
    dinvb, xs = _tc1_call(degp, degp, x)
    p = _spmm_call()(src3, dst3, xs, zeros_c).reshape(NC, NPAD, D)
    (gs,) = _tc2_call(p, p, xs, dinvb, W1, b1.reshape(1, HID), W2)
    q = _spmm_call()(src3, dst3, gs, zeros_c).reshape(NC, NPAD, D)
    (out,) = _tc3_call(q, q, gs, dinvb, b2.reshape(1, D))
    return out
